# Initial kernel scaffold; baseline (speedup 1.0000x reference)
import functools

import jax
import jax.numpy as jnp
from jax import lax
from jax.experimental import pallas as pl
from jax.experimental.pallas import tpu as pltpu
from jax.experimental.pallas import tpu_sc as plsc

NC = 2
NS = 16
NW = NC * NS
LANES = 16
BN_EPS = 1e-5


def _silu(x):
    return x * jax.nn.sigmoid(x)


def _nodeproj_body(hv_ref, w1a_ref, w1b_ref, b1_ref, a_ref, b_ref):
    hv = hv_ref[...]
    a_ref[...] = (jnp.dot(hv, w1a_ref[...], preferred_element_type=jnp.float32)
                  + b1_ref[...])
    b_ref[...] = jnp.dot(hv, w1b_ref[...], preferred_element_type=jnp.float32)


def _nodeproj(h_V, W1a, W1b, b1):
    n, d = h_V.shape
    return pl.pallas_call(
        _nodeproj_body,
        out_shape=(jax.ShapeDtypeStruct((n, d), jnp.float32),
                   jax.ShapeDtypeStruct((n, d), jnp.float32)),
    )(h_V, W1a, W1b, b1)


def _make_gather(E, D, C):
    EPW = E // NW
    NCHUNK = EPW // C
    mesh = plsc.VectorSubcoreMesh(core_axis_name="c", subcore_axis_name="s")

    @functools.partial(
        pl.kernel,
        out_type=jax.ShapeDtypeStruct((E, D), jnp.float32),
        mesh=mesh,
        scratch_types=[
            pltpu.VMEM((C,), jnp.int32),
            pltpu.VMEM((C,), jnp.int32),
            pltpu.VMEM((C, D), jnp.float32),
            pltpu.VMEM((C, D), jnp.float32),
            pltpu.SemaphoreType.DMA,
            pltpu.SemaphoreType.DMA,
        ],
    )
    def gather_k(a_hbm, b_hbm, src_hbm, dst_hbm, out_hbm,
                 idx_s, idx_d, rows_a, rows_b, sem_a, sem_b):
        wid = lax.axis_index("s") * NC + lax.axis_index("c")
        base = wid * EPW

        def body(i, carry):
            off = base + i * C
            pltpu.sync_copy(src_hbm.at[pl.ds(off, C)], idx_s)
            pltpu.sync_copy(dst_hbm.at[pl.ds(off, C)], idx_d)
            cpa = pltpu.async_copy(a_hbm.at[idx_s], rows_a, sem_a)
            cpb = pltpu.async_copy(b_hbm.at[idx_d], rows_b, sem_b)
            cpa.wait()
            cpb.wait()

            def add_row(c, carry2):
                for j in range(D // LANES):
                    sl = pl.ds(j * LANES, LANES)
                    rows_a[c, sl] = rows_a[c, sl] + rows_b[c, sl]
                return carry2

            lax.fori_loop(0, C, add_row, 0, unroll=2)
            pltpu.sync_copy(rows_a, out_hbm.at[pl.ds(off, C)])
            return carry

        lax.fori_loop(0, NCHUNK, body, 0)

    return gather_k


def _edgemlp_body(g_ref, w2_ref, b2_ref, o_ref):
    x = _silu(g_ref[...])
    y = jnp.dot(x, w2_ref[...], preferred_element_type=jnp.float32) + b2_ref[...]
    o_ref[...] = _silu(y)


def _edgemlp(G, W2, b2, BE):
    E, D = G.shape
    grid = (E // BE,)
    return pl.pallas_call(
        _edgemlp_body,
        grid=grid,
        in_specs=[
            pl.BlockSpec((BE, D), lambda i: (i, 0)),
            pl.BlockSpec((D, D), lambda i: (0, 0)),
            pl.BlockSpec((D,), lambda i: (0,)),
        ],
        out_specs=pl.BlockSpec((BE, D), lambda i: (i, 0)),
        out_shape=jax.ShapeDtypeStruct((E, D), jnp.float32),
    )(G, W2, b2)


def _make_scatter(E, D, C, NP):
    EPW = E // NW
    NCHUNK = EPW // C
    RPS = NP // NS
    mesh = plsc.VectorSubcoreMesh(core_axis_name="c", subcore_axis_name="s")

    @functools.partial(
        pl.kernel,
        out_type=(jax.ShapeDtypeStruct((NC, NP, D), jnp.float32),
                  jax.ShapeDtypeStruct((NC, NP), jnp.float32)),
        mesh=mesh,
        scratch_types=[
            pltpu.VMEM((C,), jnp.int32),
            pltpu.VMEM((C, D), jnp.float32),
            pltpu.VMEM((C,), jnp.float32),
            pltpu.VMEM((RPS,), jnp.float32),
            pltpu.VMEM_SHARED((NP, D), jnp.float32),
            pltpu.VMEM_SHARED((NP,), jnp.float32),
        ],
    )
    def scatter_k(dh_hbm, src_hbm, s_out, cnt_out,
                  idx_v, rows_v, ones_v, zcnt_v, acc_sh, cnt_sh):
        cid = lax.axis_index("c")
        sid = lax.axis_index("s")
        wid = sid * NC + cid

        zeros16 = jnp.zeros((LANES,), jnp.float32)
        ones16 = jnp.ones((LANES,), jnp.float32)

        def zrow(c, carry):
            for j in range(D // LANES):
                rows_v[c, pl.ds(j * LANES, LANES)] = zeros16
            return carry

        lax.fori_loop(0, C, zrow, 0, unroll=2)
        for j in range(C // LANES):
            ones_v[pl.ds(j * LANES, LANES)] = ones16

        def zc(i, carry):
            zcnt_v[pl.ds(i * LANES, LANES)] = zeros16
            return carry

        lax.fori_loop(0, RPS // LANES, zc, 0, unroll=2)

        row0 = sid * RPS

        def zbody(k, carry):
            pltpu.sync_copy(rows_v, acc_sh.at[pl.ds(row0 + k * C, C)])
            return carry

        lax.fori_loop(0, RPS // C, zbody, 0)
        pltpu.sync_copy(zcnt_v, cnt_sh.at[pl.ds(row0, RPS)])
        plsc.subcore_barrier()

        base = wid * EPW

        def body(i, carry):
            off = base + i * C
            pltpu.sync_copy(src_hbm.at[pl.ds(off, C)], idx_v)
            pltpu.sync_copy(dh_hbm.at[pl.ds(off, C)], rows_v)
            pltpu.sync_copy(rows_v, acc_sh.at[idx_v], add=True)
            pltpu.sync_copy(ones_v, cnt_sh.at[idx_v], add=True)
            return carry

        lax.fori_loop(0, NCHUNK, body, 0)
        plsc.subcore_barrier()

        pltpu.sync_copy(acc_sh.at[pl.ds(row0, RPS)],
                        s_out.at[cid, pl.ds(row0, RPS)])
        pltpu.sync_copy(cnt_sh.at[pl.ds(row0, RPS)],
                        cnt_out.at[cid, pl.ds(row0, RPS)])

    return scatter_k


def _final_body(hv_ref, s_ref, cnt_ref, w3_ref, b3_ref, dw1_ref, db1_ref,
                dw2_ref, db2_ref, g0_ref, be0_ref, g1_ref, be1_ref, o_ref):
    s = s_ref[0] + s_ref[1]
    cnt = cnt_ref[0] + cnt_ref[1]
    m = s / jnp.maximum(cnt, 1.0)[:, None]
    dh = jnp.dot(m, w3_ref[...], preferred_element_type=jnp.float32) + b3_ref[...]
    t = hv_ref[...] + dh
    mu = jnp.mean(t, axis=0)
    var = jnp.mean((t - mu) ** 2, axis=0)
    h = (t - mu) * lax.rsqrt(var + BN_EPS) * g0_ref[...] + be0_ref[...]
    u = _silu(jnp.dot(h, dw1_ref[...], preferred_element_type=jnp.float32)
              + db1_ref[...])
    d2 = jnp.dot(u, dw2_ref[...], preferred_element_type=jnp.float32) + db2_ref[...]
    t2 = h + d2
    mu2 = jnp.mean(t2, axis=0)
    var2 = jnp.mean((t2 - mu2) ** 2, axis=0)
    o_ref[...] = (t2 - mu2) * lax.rsqrt(var2 + BN_EPS) * g1_ref[...] + be1_ref[...]


def _final(h_V, s2, c2, W3, b3, Dw1, Db1, Dw2, Db2, g0, be0, g1, be1):
    n, d = h_V.shape
    return pl.pallas_call(
        _final_body,
        out_shape=jax.ShapeDtypeStruct((n, d), jnp.float32),
    )(h_V, s2, c2, W3, b3, Dw1, Db1, Dw2, Db2, g0, be0, g1, be1)


def kernel(h_V, edge_idx, batch_id, W1, b1, W2, b2, W3, b3,
           Dw1, Db1, Dw2, Db2, g0, be0, g1, be1):
    n, d = h_V.shape
    E = edge_idx.shape[1]
    src = edge_idx[0]
    dst = edge_idx[1]

    NP = -(-n // (NS * 8)) * (NS * 8)
    C = 80

    A, B = _nodeproj(h_V, W1[:d], W1[d:], b1)
    G = _make_gather(E, d, C)(A, B, src, dst)
    dh2 = _edgemlp(G, W2, b2, BE=4000)
    s2, c2 = _make_scatter(E, d, C, NP)(dh2, src)
    return _final(h_V, s2[:, :n], c2[:, :n], W3, b3,
                  Dw1, Db1, Dw2, Db2, g0, be0, g1, be1)


# speedup vs baseline: 2.5576x; 2.5576x over previous
import functools

import jax
import jax.numpy as jnp
from jax import lax
from jax.experimental import pallas as pl
from jax.experimental.pallas import tpu as pltpu
from jax.experimental.pallas import tpu_sc as plsc

NC = 2
NS = 16
NW = NC * NS
LANES = 16
BN_EPS = 1e-5


def _silu(x):
    return x * jax.nn.sigmoid(x)


def _nodeproj_body(hv_ref, w1a_ref, w1b_ref, b1_ref, a_ref, b_ref):
    hv = hv_ref[...]
    a_ref[...] = (jnp.dot(hv, w1a_ref[...], preferred_element_type=jnp.float32)
                  + b1_ref[...])
    b_ref[...] = jnp.dot(hv, w1b_ref[...], preferred_element_type=jnp.float32)


def _nodeproj(h_V, W1a, W1b, b1):
    n, d = h_V.shape
    return pl.pallas_call(
        _nodeproj_body,
        out_shape=(jax.ShapeDtypeStruct((n, d), jnp.float32),
                   jax.ShapeDtypeStruct((n, d), jnp.float32)),
    )(h_V, W1a, W1b, b1)


def _make_gather(E, D, C):
    EPW = E // NW
    NCHUNK = EPW // C
    mesh = plsc.VectorSubcoreMesh(core_axis_name="c", subcore_axis_name="s", num_cores=NC, num_subcores=NS)

    @functools.partial(
        pl.kernel,
        out_type=jax.ShapeDtypeStruct((E, D), jnp.float32),
        mesh=mesh,
        scratch_types=[
            pltpu.VMEM((C,), jnp.int32),
            pltpu.VMEM((C,), jnp.int32),
            pltpu.VMEM((C, D), jnp.float32),
            pltpu.VMEM((C, D), jnp.float32),
            pltpu.SemaphoreType.DMA,
            pltpu.SemaphoreType.DMA,
        ],
    )
    def gather_k(a_hbm, b_hbm, src_hbm, dst_hbm, out_hbm,
                 idx_s, idx_d, rows_a, rows_b, sem_a, sem_b):
        wid = lax.axis_index("s") * NC + lax.axis_index("c")
        base = wid * EPW

        def body(i, carry):
            off = base + i * C
            pltpu.sync_copy(src_hbm.at[pl.ds(off, C)], idx_s)
            pltpu.sync_copy(dst_hbm.at[pl.ds(off, C)], idx_d)
            cpa = pltpu.async_copy(a_hbm.at[idx_s], rows_a, sem_a)
            cpb = pltpu.async_copy(b_hbm.at[idx_d], rows_b, sem_b)
            cpa.wait()
            cpb.wait()

            def add_row(c, carry2):
                for j in range(D // LANES):
                    sl = pl.ds(j * LANES, LANES)
                    rows_a[c, sl] = rows_a[c, sl] + rows_b[c, sl]
                return carry2

            lax.fori_loop(0, C, add_row, 0, unroll=2)
            pltpu.sync_copy(rows_a, out_hbm.at[pl.ds(off, C)])
            return carry

        lax.fori_loop(0, NCHUNK, body, 0)

    return gather_k


def _edgemlp_body(g_ref, w2_ref, b2_ref, o_ref):
    x = _silu(g_ref[...])
    y = jnp.dot(x, w2_ref[...], preferred_element_type=jnp.float32) + b2_ref[...]
    o_ref[...] = _silu(y)


def _edgemlp(G, W2, b2, BE):
    E, D = G.shape
    grid = (E // BE,)
    return pl.pallas_call(
        _edgemlp_body,
        grid=grid,
        in_specs=[
            pl.BlockSpec((BE, D), lambda i: (i, 0)),
            pl.BlockSpec((D, D), lambda i: (0, 0)),
            pl.BlockSpec((D,), lambda i: (0,)),
        ],
        out_specs=pl.BlockSpec((BE, D), lambda i: (i, 0)),
        out_shape=jax.ShapeDtypeStruct((E, D), jnp.float32),
    )(G, W2, b2)


def _make_scatter(E, D, C, NP):
    EPW = E // NW
    NCHUNK = EPW // C
    RPS = NP // NS
    mesh = plsc.VectorSubcoreMesh(core_axis_name="c", subcore_axis_name="s", num_cores=NC, num_subcores=NS)

    @functools.partial(
        pl.kernel,
        out_type=(jax.ShapeDtypeStruct((NC, NP, D), jnp.float32),
                  jax.ShapeDtypeStruct((NP,), jnp.float32),
                  jax.ShapeDtypeStruct((NP,), jnp.float32)),
        mesh=mesh,
        scratch_types=[
            pltpu.VMEM((C,), jnp.int32),
            pltpu.VMEM((C, D), jnp.float32),
            pltpu.VMEM((C,), jnp.float32),
            pltpu.VMEM((RPS,), jnp.float32),
            pltpu.VMEM_SHARED((NP, D), jnp.float32),
            pltpu.VMEM_SHARED((NP,), jnp.float32),
        ],
    )
    def scatter_k(dh_hbm, src_hbm, s_out, cnt0_out, cnt1_out,
                  idx_v, rows_v, ones_v, zcnt_v, acc_sh, cnt_sh):
        cid = lax.axis_index("c")
        sid = lax.axis_index("s")
        wid = sid * NC + cid

        zeros16 = jnp.zeros((LANES,), jnp.float32)
        ones16 = jnp.ones((LANES,), jnp.float32)

        def zrow(c, carry):
            for j in range(D // LANES):
                rows_v[c, pl.ds(j * LANES, LANES)] = zeros16
            return carry

        lax.fori_loop(0, C, zrow, 0, unroll=2)
        for j in range(C // LANES):
            ones_v[pl.ds(j * LANES, LANES)] = ones16

        def zc(i, carry):
            zcnt_v[pl.ds(i * LANES, LANES)] = zeros16
            return carry

        lax.fori_loop(0, RPS // LANES, zc, 0, unroll=2)

        row0 = sid * RPS

        def zbody(k, carry):
            pltpu.sync_copy(rows_v, acc_sh.at[pl.ds(row0 + k * C, C)])
            return carry

        lax.fori_loop(0, RPS // C, zbody, 0)
        pltpu.sync_copy(zcnt_v, cnt_sh.at[pl.ds(row0, RPS)])
        plsc.subcore_barrier()

        base = wid * EPW

        def body(i, carry):
            off = base + i * C
            pltpu.sync_copy(src_hbm.at[pl.ds(off, C)], idx_v)
            pltpu.sync_copy(dh_hbm.at[pl.ds(off, C)], rows_v)
            pltpu.sync_copy(rows_v, acc_sh.at[idx_v], add=True)
            pltpu.sync_copy(ones_v, cnt_sh.at[idx_v], add=True)
            return carry

        lax.fori_loop(0, NCHUNK, body, 0)
        plsc.subcore_barrier()

        pltpu.sync_copy(acc_sh.at[pl.ds(row0, RPS)],
                        s_out.at[cid, pl.ds(row0, RPS)])

        @pl.when(cid == 0)
        def _():
            pltpu.sync_copy(cnt_sh.at[pl.ds(row0, RPS)],
                            cnt0_out.at[pl.ds(row0, RPS)])

        @pl.when(cid == 1)
        def _():
            pltpu.sync_copy(cnt_sh.at[pl.ds(row0, RPS)],
                            cnt1_out.at[pl.ds(row0, RPS)])

    return scatter_k


def _final_body(hv_ref, s_ref, c0_ref, c1_ref, w3_ref, b3_ref, dw1_ref, db1_ref,
                dw2_ref, db2_ref, g0_ref, be0_ref, g1_ref, be1_ref, o_ref):
    s = s_ref[0] + s_ref[1]
    cnt = c0_ref[...] + c1_ref[...]
    m = s / jnp.maximum(cnt, 1.0)[:, None]
    dh = jnp.dot(m, w3_ref[...], preferred_element_type=jnp.float32) + b3_ref[...]
    t = hv_ref[...] + dh
    mu = jnp.mean(t, axis=0)
    var = jnp.mean((t - mu) ** 2, axis=0)
    h = (t - mu) * lax.rsqrt(var + BN_EPS) * g0_ref[...] + be0_ref[...]
    u = _silu(jnp.dot(h, dw1_ref[...], preferred_element_type=jnp.float32)
              + db1_ref[...])
    d2 = jnp.dot(u, dw2_ref[...], preferred_element_type=jnp.float32) + db2_ref[...]
    t2 = h + d2
    mu2 = jnp.mean(t2, axis=0)
    var2 = jnp.mean((t2 - mu2) ** 2, axis=0)
    o_ref[...] = (t2 - mu2) * lax.rsqrt(var2 + BN_EPS) * g1_ref[...] + be1_ref[...]


def _final(h_V, s2, c0, c1, W3, b3, Dw1, Db1, Dw2, Db2, g0, be0, g1, be1):
    n, d = h_V.shape
    return pl.pallas_call(
        _final_body,
        out_shape=jax.ShapeDtypeStruct((n, d), jnp.float32),
    )(h_V, s2, c0, c1, W3, b3, Dw1, Db1, Dw2, Db2, g0, be0, g1, be1)


def kernel(h_V, edge_idx, batch_id, W1, b1, W2, b2, W3, b3,
           Dw1, Db1, Dw2, Db2, g0, be0, g1, be1):
    n, d = h_V.shape
    E = edge_idx.shape[1]
    src = edge_idx[0]
    dst = edge_idx[1]

    NP = -(-n // (NS * LANES)) * (NS * LANES)
    C = 80

    A, B = _nodeproj(h_V, W1[:d], W1[d:], b1)
    G = _make_gather(E, d, C)(A, B, src, dst)
    dh2 = _edgemlp(G, W2, b2, BE=4000)
    s2, c0, c1 = _make_scatter(E, d, C, NP)(dh2, src)
    return _final(h_V, s2[:, :n], c0[:n], c1[:n], W3, b3,
                  Dw1, Db1, Dw2, Db2, g0, be0, g1, be1)


# speedup vs baseline: 4.6106x; 1.8027x over previous
import functools

import jax
import jax.numpy as jnp
from jax import lax
from jax.experimental import pallas as pl
from jax.experimental.pallas import tpu as pltpu
from jax.experimental.pallas import tpu_sc as plsc

NC = 2
NS = 16
NW = NC * NS
LANES = 16
BN_EPS = 1e-5


def _silu(x):
    return x * jax.nn.sigmoid(x)


def _nodeproj_body(hv_ref, w1a_ref, w1b_ref, b1_ref, a_ref, b_ref):
    hv = hv_ref[...]
    a_ref[...] = (jnp.dot(hv, w1a_ref[...], preferred_element_type=jnp.float32)
                  + b1_ref[...])
    b_ref[...] = jnp.dot(hv, w1b_ref[...], preferred_element_type=jnp.float32)


def _nodeproj(h_V, W1a, W1b, b1):
    n, d = h_V.shape
    return pl.pallas_call(
        _nodeproj_body,
        out_shape=(jax.ShapeDtypeStruct((n, d), jnp.float32),
                   jax.ShapeDtypeStruct((n, d), jnp.float32)),
    )(h_V, W1a, W1b, b1)


def _make_gather(E, D, C):
    EPW = E // NW
    NCHUNK = EPW // C
    NPAIR = NCHUNK // 2
    mesh = plsc.VectorSubcoreMesh(core_axis_name="c", subcore_axis_name="s", num_cores=NC, num_subcores=NS)

    @functools.partial(
        pl.kernel,
        out_type=jax.ShapeDtypeStruct((E, D), jnp.float32),
        mesh=mesh,
        scratch_types=[
            pltpu.VMEM((EPW,), jnp.int32),
            pltpu.VMEM((EPW,), jnp.int32),
            pltpu.VMEM((C, D), jnp.float32),
            pltpu.VMEM((C, D), jnp.float32),
            pltpu.VMEM((C, D), jnp.float32),
            pltpu.VMEM((C, D), jnp.float32),
            pltpu.VMEM((C, D), jnp.float32),
            pltpu.VMEM((C, D), jnp.float32),
            pltpu.SemaphoreType.DMA,
            pltpu.SemaphoreType.DMA,
            pltpu.SemaphoreType.DMA,
            pltpu.SemaphoreType.DMA,
            pltpu.SemaphoreType.DMA,
            pltpu.SemaphoreType.DMA,
        ],
    )
    def gather_k(a_hbm, b_hbm, src_hbm, dst_hbm, out_hbm,
                 idx_s, idx_d, a0, a1, b0, b1, o0, o1,
                 sga0, sga1, sgb0, sgb1, swb0, swb1):
        wid = lax.axis_index("s") * NC + lax.axis_index("c")
        base = wid * EPW

        pltpu.sync_copy(src_hbm.at[pl.ds(base, EPW)], idx_s)
        pltpu.sync_copy(dst_hbm.at[pl.ds(base, EPW)], idx_d)

        def fire(c, abuf, bbuf, sa, sb):
            pltpu.async_copy(a_hbm.at[idx_s.at[pl.ds(c * C, C)]], abuf, sa)
            pltpu.async_copy(b_hbm.at[idx_d.at[pl.ds(c * C, C)]], bbuf, sb)

        def wait_g(c, abuf, bbuf, sa, sb):
            pltpu.make_async_copy(a_hbm.at[idx_s.at[pl.ds(c * C, C)]], abuf, sa).wait()
            pltpu.make_async_copy(b_hbm.at[idx_d.at[pl.ds(c * C, C)]], bbuf, sb).wait()

        def wait_wb(obuf, swb):
            pltpu.make_async_copy(obuf, out_hbm.at[pl.ds(base, C)], swb).wait()

        def add(abuf, bbuf, obuf):
            @plsc.parallel_loop(0, C, unroll=4)
            def _(r):
                for j in range(D // LANES):
                    sl = pl.ds(j * LANES, LANES)
                    obuf[r, sl] = abuf[r, sl] + bbuf[r, sl]

        def step(c, i2, abuf, bbuf, obuf, sa, sb, swb):
            wait_g(c, abuf, bbuf, sa, sb)

            @pl.when(i2 > 0)
            def _():
                wait_wb(obuf, swb)

            add(abuf, bbuf, obuf)
            pltpu.async_copy(obuf, out_hbm.at[pl.ds(base + c * C, C)], swb)

            @pl.when(c + 2 < NCHUNK)
            def _():
                fire(c + 2, abuf, bbuf, sa, sb)

        fire(0, a0, b0, sga0, sgb0)
        fire(1, a1, b1, sga1, sgb1)

        def pair(i2, carry):
            c0 = 2 * i2
            step(c0, i2, a0, b0, o0, sga0, sgb0, swb0)
            step(c0 + 1, i2, a1, b1, o1, sga1, sgb1, swb1)
            return carry

        lax.fori_loop(0, NPAIR, pair, 0)

        if NCHUNK % 2 == 1:
            c = NCHUNK - 1
            wait_g(c, a0, b0, sga0, sgb0)
            if NPAIR > 0:
                wait_wb(o0, swb0)
            add(a0, b0, o0)
            pltpu.async_copy(o0, out_hbm.at[pl.ds(base + c * C, C)], swb0)

        wait_wb(o0, swb0)
        if NPAIR > 0:
            wait_wb(o1, swb1)

    return gather_k


def _edgemlp_body(g_ref, w2_ref, b2_ref, o_ref):
    x = _silu(g_ref[...])
    y = jnp.dot(x, w2_ref[...], preferred_element_type=jnp.float32) + b2_ref[...]
    o_ref[...] = _silu(y)


def _edgemlp(G, W2, b2, BE):
    E, D = G.shape
    grid = (E // BE,)
    return pl.pallas_call(
        _edgemlp_body,
        grid=grid,
        in_specs=[
            pl.BlockSpec((BE, D), lambda i: (i, 0)),
            pl.BlockSpec((D, D), lambda i: (0, 0)),
            pl.BlockSpec((D,), lambda i: (0,)),
        ],
        out_specs=pl.BlockSpec((BE, D), lambda i: (i, 0)),
        out_shape=jax.ShapeDtypeStruct((E, D), jnp.float32),
    )(G, W2, b2)


def _make_scatter(E, D, C, NP):
    EPW = E // NW
    NCHUNK = EPW // C
    RPS = NP // NS
    mesh = plsc.VectorSubcoreMesh(core_axis_name="c", subcore_axis_name="s", num_cores=NC, num_subcores=NS)

    @functools.partial(
        pl.kernel,
        out_type=(jax.ShapeDtypeStruct((NC, NP, D), jnp.float32),
                  jax.ShapeDtypeStruct((NP,), jnp.float32),
                  jax.ShapeDtypeStruct((NP,), jnp.float32)),
        mesh=mesh,
        scratch_types=[
            pltpu.VMEM((C,), jnp.int32),
            pltpu.VMEM((C, D), jnp.float32),
            pltpu.VMEM((C,), jnp.float32),
            pltpu.VMEM((RPS,), jnp.float32),
            pltpu.VMEM_SHARED((NP, D), jnp.float32),
            pltpu.VMEM_SHARED((NP,), jnp.float32),
        ],
    )
    def scatter_k(dh_hbm, src_hbm, s_out, cnt0_out, cnt1_out,
                  idx_v, rows_v, ones_v, zcnt_v, acc_sh, cnt_sh):
        cid = lax.axis_index("c")
        sid = lax.axis_index("s")
        wid = sid * NC + cid

        zeros16 = jnp.zeros((LANES,), jnp.float32)
        ones16 = jnp.ones((LANES,), jnp.float32)

        def zrow(c, carry):
            for j in range(D // LANES):
                rows_v[c, pl.ds(j * LANES, LANES)] = zeros16
            return carry

        lax.fori_loop(0, C, zrow, 0, unroll=2)
        for j in range(C // LANES):
            ones_v[pl.ds(j * LANES, LANES)] = ones16

        def zc(i, carry):
            zcnt_v[pl.ds(i * LANES, LANES)] = zeros16
            return carry

        lax.fori_loop(0, RPS // LANES, zc, 0, unroll=2)

        row0 = sid * RPS

        def zbody(k, carry):
            pltpu.sync_copy(rows_v, acc_sh.at[pl.ds(row0 + k * C, C)])
            return carry

        lax.fori_loop(0, RPS // C, zbody, 0)
        pltpu.sync_copy(zcnt_v, cnt_sh.at[pl.ds(row0, RPS)])
        plsc.subcore_barrier()

        base = wid * EPW

        def body(i, carry):
            off = base + i * C
            pltpu.sync_copy(src_hbm.at[pl.ds(off, C)], idx_v)
            pltpu.sync_copy(dh_hbm.at[pl.ds(off, C)], rows_v)
            pltpu.sync_copy(rows_v, acc_sh.at[idx_v], add=True)
            pltpu.sync_copy(ones_v, cnt_sh.at[idx_v], add=True)
            return carry

        lax.fori_loop(0, NCHUNK, body, 0)
        plsc.subcore_barrier()

        pltpu.sync_copy(acc_sh.at[pl.ds(row0, RPS)],
                        s_out.at[cid, pl.ds(row0, RPS)])

        @pl.when(cid == 0)
        def _():
            pltpu.sync_copy(cnt_sh.at[pl.ds(row0, RPS)],
                            cnt0_out.at[pl.ds(row0, RPS)])

        @pl.when(cid == 1)
        def _():
            pltpu.sync_copy(cnt_sh.at[pl.ds(row0, RPS)],
                            cnt1_out.at[pl.ds(row0, RPS)])

    return scatter_k


def _final_body(hv_ref, s_ref, c0_ref, c1_ref, w3_ref, b3_ref, dw1_ref, db1_ref,
                dw2_ref, db2_ref, g0_ref, be0_ref, g1_ref, be1_ref, o_ref):
    s = s_ref[0] + s_ref[1]
    cnt = c0_ref[...] + c1_ref[...]
    m = s / jnp.maximum(cnt, 1.0)[:, None]
    dh = jnp.dot(m, w3_ref[...], preferred_element_type=jnp.float32) + b3_ref[...]
    t = hv_ref[...] + dh
    mu = jnp.mean(t, axis=0)
    var = jnp.mean((t - mu) ** 2, axis=0)
    h = (t - mu) * lax.rsqrt(var + BN_EPS) * g0_ref[...] + be0_ref[...]
    u = _silu(jnp.dot(h, dw1_ref[...], preferred_element_type=jnp.float32)
              + db1_ref[...])
    d2 = jnp.dot(u, dw2_ref[...], preferred_element_type=jnp.float32) + db2_ref[...]
    t2 = h + d2
    mu2 = jnp.mean(t2, axis=0)
    var2 = jnp.mean((t2 - mu2) ** 2, axis=0)
    o_ref[...] = (t2 - mu2) * lax.rsqrt(var2 + BN_EPS) * g1_ref[...] + be1_ref[...]


def _final(h_V, s2, c0, c1, W3, b3, Dw1, Db1, Dw2, Db2, g0, be0, g1, be1):
    n, d = h_V.shape
    return pl.pallas_call(
        _final_body,
        out_shape=jax.ShapeDtypeStruct((n, d), jnp.float32),
    )(h_V, s2, c0, c1, W3, b3, Dw1, Db1, Dw2, Db2, g0, be0, g1, be1)


def kernel(h_V, edge_idx, batch_id, W1, b1, W2, b2, W3, b3,
           Dw1, Db1, Dw2, Db2, g0, be0, g1, be1):
    n, d = h_V.shape
    E = edge_idx.shape[1]
    src = edge_idx[0]
    dst = edge_idx[1]

    NP = -(-n // (NS * LANES)) * (NS * LANES)
    C = 80

    A, B = _nodeproj(h_V, W1[:d], W1[d:], b1)
    G = _make_gather(E, d, C)(A, B, src, dst)
    dh2 = _edgemlp(G, W2, b2, BE=4000)
    s2, c0, c1 = _make_scatter(E, d, C, NP)(dh2, src)
    return _final(h_V, s2[:, :n], c0[:n], c1[:n], W3, b3,
                  Dw1, Db1, Dw2, Db2, g0, be0, g1, be1)


# speedup vs baseline: 5.9455x; 1.2895x over previous
import functools

import jax
import jax.numpy as jnp
from jax import lax
from jax.experimental import pallas as pl
from jax.experimental.pallas import tpu as pltpu
from jax.experimental.pallas import tpu_sc as plsc

NC = 2
NS = 16
NW = NC * NS
LANES = 16
BN_EPS = 1e-5


def _silu(x):
    return x * jax.nn.sigmoid(x)


def _nodeproj_body(hv_ref, w1a_ref, w1b_ref, b1_ref, a_ref, b_ref):
    hv = hv_ref[...]
    a_ref[...] = (jnp.dot(hv, w1a_ref[...], preferred_element_type=jnp.float32)
                  + b1_ref[...])
    b_ref[...] = jnp.dot(hv, w1b_ref[...], preferred_element_type=jnp.float32)


def _nodeproj(h_V, W1a, W1b, b1):
    n, d = h_V.shape
    return pl.pallas_call(
        _nodeproj_body,
        out_shape=(jax.ShapeDtypeStruct((n, d), jnp.float32),
                   jax.ShapeDtypeStruct((n, d), jnp.float32)),
    )(h_V, W1a, W1b, b1)


def _make_gather(E, D, C):
    EPW = E // NW
    NCHUNK = EPW // C
    NPAIR = NCHUNK // 2
    mesh = plsc.VectorSubcoreMesh(core_axis_name="c", subcore_axis_name="s", num_cores=NC, num_subcores=NS)

    @functools.partial(
        pl.kernel,
        out_type=jax.ShapeDtypeStruct((E, D), jnp.float32),
        mesh=mesh,
        scratch_types=[
            pltpu.VMEM((EPW,), jnp.int32),
            pltpu.VMEM((EPW,), jnp.int32),
            pltpu.VMEM((C, D), jnp.float32),
            pltpu.VMEM((C, D), jnp.float32),
            pltpu.VMEM((C, D), jnp.float32),
            pltpu.VMEM((C, D), jnp.float32),
            pltpu.VMEM((C, D), jnp.float32),
            pltpu.VMEM((C, D), jnp.float32),
            pltpu.SemaphoreType.DMA,
            pltpu.SemaphoreType.DMA,
            pltpu.SemaphoreType.DMA,
            pltpu.SemaphoreType.DMA,
            pltpu.SemaphoreType.DMA,
            pltpu.SemaphoreType.DMA,
        ],
    )
    def gather_k(a_hbm, b_hbm, src_hbm, dst_hbm, out_hbm,
                 idx_s, idx_d, a0, a1, b0, b1, o0, o1,
                 sga0, sga1, sgb0, sgb1, swb0, swb1):
        wid = lax.axis_index("s") * NC + lax.axis_index("c")
        base = wid * EPW

        pltpu.sync_copy(src_hbm.at[pl.ds(base, EPW)], idx_s)
        pltpu.sync_copy(dst_hbm.at[pl.ds(base, EPW)], idx_d)

        def fire(c, abuf, bbuf, sa, sb):
            pltpu.async_copy(a_hbm.at[idx_s.at[pl.ds(c * C, C)]], abuf, sa)
            pltpu.async_copy(b_hbm.at[idx_d.at[pl.ds(c * C, C)]], bbuf, sb)

        def wait_g(c, abuf, bbuf, sa, sb):
            pltpu.make_async_copy(a_hbm.at[idx_s.at[pl.ds(c * C, C)]], abuf, sa).wait()
            pltpu.make_async_copy(b_hbm.at[idx_d.at[pl.ds(c * C, C)]], bbuf, sb).wait()

        def wait_wb(obuf, swb):
            pltpu.make_async_copy(obuf, out_hbm.at[pl.ds(base, C)], swb).wait()

        def add(abuf, bbuf, obuf):
            @plsc.parallel_loop(0, C, unroll=4)
            def _(r):
                for j in range(D // LANES):
                    sl = pl.ds(j * LANES, LANES)
                    obuf[r, sl] = abuf[r, sl] + bbuf[r, sl]

        def step(c, i2, abuf, bbuf, obuf, sa, sb, swb):
            wait_g(c, abuf, bbuf, sa, sb)

            @pl.when(i2 > 0)
            def _():
                wait_wb(obuf, swb)

            add(abuf, bbuf, obuf)
            pltpu.async_copy(obuf, out_hbm.at[pl.ds(base + c * C, C)], swb)

            @pl.when(c + 2 < NCHUNK)
            def _():
                fire(c + 2, abuf, bbuf, sa, sb)

        fire(0, a0, b0, sga0, sgb0)
        fire(1, a1, b1, sga1, sgb1)

        def pair(i2, carry):
            c0 = 2 * i2
            step(c0, i2, a0, b0, o0, sga0, sgb0, swb0)
            step(c0 + 1, i2, a1, b1, o1, sga1, sgb1, swb1)
            return carry

        lax.fori_loop(0, NPAIR, pair, 0)

        if NCHUNK % 2 == 1:
            c = NCHUNK - 1
            wait_g(c, a0, b0, sga0, sgb0)
            if NPAIR > 0:
                wait_wb(o0, swb0)
            add(a0, b0, o0)
            pltpu.async_copy(o0, out_hbm.at[pl.ds(base + c * C, C)], swb0)

        wait_wb(o0, swb0)
        if NPAIR > 0:
            wait_wb(o1, swb1)

    return gather_k


def _edgemlp_body(g_ref, w2_ref, b2_ref, o_ref):
    x = _silu(g_ref[...])
    y = jnp.dot(x, w2_ref[...], preferred_element_type=jnp.float32) + b2_ref[...]
    o_ref[...] = _silu(y)


def _edgemlp(G, W2, b2, BE):
    E, D = G.shape
    grid = (E // BE,)
    return pl.pallas_call(
        _edgemlp_body,
        grid=grid,
        in_specs=[
            pl.BlockSpec((BE, D), lambda i: (i, 0)),
            pl.BlockSpec((D, D), lambda i: (0, 0)),
            pl.BlockSpec((D,), lambda i: (0,)),
        ],
        out_specs=pl.BlockSpec((BE, D), lambda i: (i, 0)),
        out_shape=jax.ShapeDtypeStruct((E, D), jnp.float32),
    )(G, W2, b2)


def _make_scatter(E, D, C, NP):
    EPW = E // NW
    NCHUNK = EPW // C
    RPS = NP // NS
    mesh = plsc.VectorSubcoreMesh(core_axis_name="c", subcore_axis_name="s", num_cores=NC, num_subcores=NS)

    @functools.partial(
        pl.kernel,
        out_type=(jax.ShapeDtypeStruct((NC, NP, D), jnp.float32),
                  jax.ShapeDtypeStruct((NP,), jnp.float32),
                  jax.ShapeDtypeStruct((NP,), jnp.float32)),
        mesh=mesh,
        scratch_types=[
            pltpu.VMEM((EPW,), jnp.int32),
            pltpu.VMEM((C, D), jnp.float32),
            pltpu.VMEM((C, D), jnp.float32),
            pltpu.VMEM((C,), jnp.float32),
            pltpu.VMEM((RPS,), jnp.float32),
            pltpu.VMEM_SHARED((NP, D), jnp.float32),
            pltpu.VMEM_SHARED((NP,), jnp.float32),
            pltpu.SemaphoreType.DMA,
            pltpu.SemaphoreType.DMA,
            pltpu.SemaphoreType.DMA,
            pltpu.SemaphoreType.DMA,
            pltpu.SemaphoreType.DMA,
            pltpu.SemaphoreType.DMA,
        ],
    )
    def scatter_k(dh_hbm, src_hbm, s_out, cnt0_out, cnt1_out,
                  idx_v, r0, r1, ones_v, zcnt_v, acc_sh, cnt_sh,
                  sl0, sl1, ss0, ss1, so0, so1):
        cid = lax.axis_index("c")
        sid = lax.axis_index("s")
        wid = sid * NC + cid
        base = wid * EPW

        idx_cp = pltpu.async_copy(src_hbm.at[pl.ds(base, EPW)], idx_v, sl0)

        zeros16 = jnp.zeros((LANES,), jnp.float32)
        ones16 = jnp.ones((LANES,), jnp.float32)

        def zrow(c, carry):
            for j in range(D // LANES):
                r0[c, pl.ds(j * LANES, LANES)] = zeros16
            return carry

        lax.fori_loop(0, C, zrow, 0, unroll=2)
        for j in range(C // LANES):
            ones_v[pl.ds(j * LANES, LANES)] = ones16

        def zc(i, carry):
            zcnt_v[pl.ds(i * LANES, LANES)] = zeros16
            return carry

        lax.fori_loop(0, RPS // LANES, zc, 0, unroll=2)

        row0 = sid * RPS

        def zbody(k, carry):
            pltpu.sync_copy(r0, acc_sh.at[pl.ds(row0 + k * C, C)])
            return carry

        lax.fori_loop(0, RPS // C, zbody, 0)
        pltpu.sync_copy(zcnt_v, cnt_sh.at[pl.ds(row0, RPS)])
        idx_cp.wait()
        plsc.subcore_barrier()

        def fire_load(c, rbuf, sl):
            pltpu.async_copy(dh_hbm.at[pl.ds(base + c * C, C)], rbuf, sl)

        def wait_load(rbuf, sl):
            pltpu.make_async_copy(dh_hbm.at[pl.ds(base, C)], rbuf, sl).wait()

        def wait_scat(c, rbuf, ss, so):
            isl = idx_v.at[pl.ds(c * C, C)]
            pltpu.make_async_copy(rbuf, acc_sh.at[isl], ss).wait()
            pltpu.make_async_copy(ones_v, cnt_sh.at[isl], so).wait()

        def step(c, rbuf, sl, ss, so):
            wait_load(rbuf, sl)
            isl = idx_v.at[pl.ds(c * C, C)]
            pltpu.async_copy(rbuf, acc_sh.at[isl], ss, add=True)
            pltpu.async_copy(ones_v, cnt_sh.at[isl], so, add=True)

            @pl.when(c + 2 < NCHUNK)
            def _():
                wait_scat(c, rbuf, ss, so)
                fire_load(c + 2, rbuf, sl)

        fire_load(0, r0, sl0)
        fire_load(1, r1, sl1)

        def pair(i2, carry):
            c0 = 2 * i2
            step(c0, r0, sl0, ss0, so0)
            step(c0 + 1, r1, sl1, ss1, so1)
            return carry

        lax.fori_loop(0, NCHUNK // 2, pair, 0)

        if NCHUNK % 2 == 1:
            step(NCHUNK - 1, r0, sl0, ss0, so0)

        wait_scat(NCHUNK - 1, r0, ss0, so0)
        if NCHUNK >= 2:
            wait_scat(NCHUNK - 2, r1, ss1, so1)
        plsc.subcore_barrier()

        pltpu.sync_copy(acc_sh.at[pl.ds(row0, RPS)],
                        s_out.at[cid, pl.ds(row0, RPS)])

        @pl.when(cid == 0)
        def _():
            pltpu.sync_copy(cnt_sh.at[pl.ds(row0, RPS)],
                            cnt0_out.at[pl.ds(row0, RPS)])

        @pl.when(cid == 1)
        def _():
            pltpu.sync_copy(cnt_sh.at[pl.ds(row0, RPS)],
                            cnt1_out.at[pl.ds(row0, RPS)])

    return scatter_k


def _final_body(hv_ref, s_ref, c0_ref, c1_ref, w3_ref, b3_ref, dw1_ref, db1_ref,
                dw2_ref, db2_ref, g0_ref, be0_ref, g1_ref, be1_ref, o_ref):
    s = s_ref[0] + s_ref[1]
    cnt = c0_ref[...] + c1_ref[...]
    m = s / jnp.maximum(cnt, 1.0)[:, None]
    dh = jnp.dot(m, w3_ref[...], preferred_element_type=jnp.float32) + b3_ref[...]
    t = hv_ref[...] + dh
    mu = jnp.mean(t, axis=0)
    var = jnp.mean((t - mu) ** 2, axis=0)
    h = (t - mu) * lax.rsqrt(var + BN_EPS) * g0_ref[...] + be0_ref[...]
    u = _silu(jnp.dot(h, dw1_ref[...], preferred_element_type=jnp.float32)
              + db1_ref[...])
    d2 = jnp.dot(u, dw2_ref[...], preferred_element_type=jnp.float32) + db2_ref[...]
    t2 = h + d2
    mu2 = jnp.mean(t2, axis=0)
    var2 = jnp.mean((t2 - mu2) ** 2, axis=0)
    o_ref[...] = (t2 - mu2) * lax.rsqrt(var2 + BN_EPS) * g1_ref[...] + be1_ref[...]


def _final(h_V, s2, c0, c1, W3, b3, Dw1, Db1, Dw2, Db2, g0, be0, g1, be1):
    n, d = h_V.shape
    return pl.pallas_call(
        _final_body,
        out_shape=jax.ShapeDtypeStruct((n, d), jnp.float32),
    )(h_V, s2, c0, c1, W3, b3, Dw1, Db1, Dw2, Db2, g0, be0, g1, be1)


def kernel(h_V, edge_idx, batch_id, W1, b1, W2, b2, W3, b3,
           Dw1, Db1, Dw2, Db2, g0, be0, g1, be1):
    n, d = h_V.shape
    E = edge_idx.shape[1]
    src = edge_idx[0]
    dst = edge_idx[1]

    NP = -(-n // (NS * LANES)) * (NS * LANES)
    C = 80

    A, B = _nodeproj(h_V, W1[:d], W1[d:], b1)
    G = _make_gather(E, d, C)(A, B, src, dst)
    dh2 = _edgemlp(G, W2, b2, BE=4000)
    s2, c0, c1 = _make_scatter(E, d, C, NP)(dh2, src)
    return _final(h_V, s2[:, :n], c0[:n], c1[:n], W3, b3,
                  Dw1, Db1, Dw2, Db2, g0, be0, g1, be1)


# speedup vs baseline: 6.1513x; 1.0346x over previous
import functools

import jax
import jax.numpy as jnp
from jax import lax
from jax.experimental import pallas as pl
from jax.experimental.pallas import tpu as pltpu
from jax.experimental.pallas import tpu_sc as plsc

NC = 2
NS = 16
NW = NC * NS
LANES = 16
BN_EPS = 1e-5


def _silu(x):
    return x * jax.nn.sigmoid(x)


def _nodeproj_body(hv_ref, w1a_ref, w1b_ref, b1_ref, a_ref, b_ref):
    hv = hv_ref[...]
    a_ref[...] = (jnp.dot(hv, w1a_ref[...], preferred_element_type=jnp.float32)
                  + b1_ref[...])
    b_ref[...] = jnp.dot(hv, w1b_ref[...], preferred_element_type=jnp.float32)


def _nodeproj(h_V, W1a, W1b, b1):
    n, d = h_V.shape
    return pl.pallas_call(
        _nodeproj_body,
        out_shape=(jax.ShapeDtypeStruct((n, d), jnp.float32),
                   jax.ShapeDtypeStruct((n, d), jnp.float32)),
    )(h_V, W1a, W1b, b1)


def _make_gather(E, D, C):
    EPW = E // NW
    NCHUNK = EPW // C
    NPAIR = NCHUNK // 2
    mesh = plsc.VectorSubcoreMesh(core_axis_name="c", subcore_axis_name="s", num_cores=NC, num_subcores=NS)

    @functools.partial(
        pl.kernel,
        out_type=jax.ShapeDtypeStruct((E, D), jnp.float32),
        mesh=mesh,
        scratch_types=[
            pltpu.VMEM((EPW,), jnp.int32),
            pltpu.VMEM((EPW,), jnp.int32),
            pltpu.VMEM((C, D), jnp.float32),
            pltpu.VMEM((C, D), jnp.float32),
            pltpu.VMEM((C, D), jnp.float32),
            pltpu.VMEM((C, D), jnp.float32),
            pltpu.VMEM((C, D), jnp.float32),
            pltpu.VMEM((C, D), jnp.float32),
            pltpu.SemaphoreType.DMA,
            pltpu.SemaphoreType.DMA,
            pltpu.SemaphoreType.DMA,
            pltpu.SemaphoreType.DMA,
            pltpu.SemaphoreType.DMA,
            pltpu.SemaphoreType.DMA,
        ],
    )
    def gather_k(a_hbm, b_hbm, ei_hbm, out_hbm,
                 idx_s, idx_d, a0, a1, b0, b1, o0, o1,
                 sga0, sga1, sgb0, sgb1, swb0, swb1):
        wid = lax.axis_index("s") * NC + lax.axis_index("c")
        base = wid * EPW

        pltpu.sync_copy(ei_hbm.at[pl.ds(base, EPW)], idx_s)
        pltpu.sync_copy(ei_hbm.at[pl.ds(E + base, EPW)], idx_d)

        def fire(c, abuf, bbuf, sa, sb):
            pltpu.async_copy(a_hbm.at[idx_s.at[pl.ds(c * C, C)]], abuf, sa)
            pltpu.async_copy(b_hbm.at[idx_d.at[pl.ds(c * C, C)]], bbuf, sb)

        def wait_g(c, abuf, bbuf, sa, sb):
            pltpu.make_async_copy(a_hbm.at[idx_s.at[pl.ds(c * C, C)]], abuf, sa).wait()
            pltpu.make_async_copy(b_hbm.at[idx_d.at[pl.ds(c * C, C)]], bbuf, sb).wait()

        def wait_wb(obuf, swb):
            pltpu.make_async_copy(obuf, out_hbm.at[pl.ds(base, C)], swb).wait()

        def add(abuf, bbuf, obuf):
            @plsc.parallel_loop(0, C, unroll=4)
            def _(r):
                for j in range(D // LANES):
                    sl = pl.ds(j * LANES, LANES)
                    obuf[r, sl] = abuf[r, sl] + bbuf[r, sl]

        def step(c, i2, abuf, bbuf, obuf, sa, sb, swb):
            wait_g(c, abuf, bbuf, sa, sb)

            @pl.when(i2 > 0)
            def _():
                wait_wb(obuf, swb)

            add(abuf, bbuf, obuf)
            pltpu.async_copy(obuf, out_hbm.at[pl.ds(base + c * C, C)], swb)

            @pl.when(c + 2 < NCHUNK)
            def _():
                fire(c + 2, abuf, bbuf, sa, sb)

        fire(0, a0, b0, sga0, sgb0)
        fire(1, a1, b1, sga1, sgb1)

        def pair(i2, carry):
            c0 = 2 * i2
            step(c0, i2, a0, b0, o0, sga0, sgb0, swb0)
            step(c0 + 1, i2, a1, b1, o1, sga1, sgb1, swb1)
            return carry

        lax.fori_loop(0, NPAIR, pair, 0)

        if NCHUNK % 2 == 1:
            c = NCHUNK - 1
            wait_g(c, a0, b0, sga0, sgb0)
            if NPAIR > 0:
                wait_wb(o0, swb0)
            add(a0, b0, o0)
            pltpu.async_copy(o0, out_hbm.at[pl.ds(base + c * C, C)], swb0)

        wait_wb(o0, swb0)
        if NPAIR > 0:
            wait_wb(o1, swb1)

    return gather_k


def _edgemlp_body(g_ref, w2_ref, b2_ref, o_ref):
    x = _silu(g_ref[...])
    y = jnp.dot(x, w2_ref[...], preferred_element_type=jnp.float32) + b2_ref[...]
    o_ref[...] = _silu(y)


def _edgemlp(G, W2, b2, BE):
    E, D = G.shape
    grid = (E // BE,)
    return pl.pallas_call(
        _edgemlp_body,
        grid=grid,
        in_specs=[
            pl.BlockSpec((BE, D), lambda i: (i, 0)),
            pl.BlockSpec((D, D), lambda i: (0, 0)),
            pl.BlockSpec((D,), lambda i: (0,)),
        ],
        out_specs=pl.BlockSpec((BE, D), lambda i: (i, 0)),
        out_shape=jax.ShapeDtypeStruct((E, D), jnp.float32),
    )(G, W2, b2)


def _make_scatter(E, D, C, NP):
    EPW = E // NW
    NCHUNK = EPW // C
    RPS = NP // NS
    mesh = plsc.VectorSubcoreMesh(core_axis_name="c", subcore_axis_name="s", num_cores=NC, num_subcores=NS)

    @functools.partial(
        pl.kernel,
        out_type=(jax.ShapeDtypeStruct((NC, NP, D), jnp.float32),
                  jax.ShapeDtypeStruct((NP,), jnp.float32),
                  jax.ShapeDtypeStruct((NP,), jnp.float32)),
        mesh=mesh,
        scratch_types=[
            pltpu.VMEM((EPW,), jnp.int32),
            pltpu.VMEM((C, D), jnp.float32),
            pltpu.VMEM((C, D), jnp.float32),
            pltpu.VMEM((C,), jnp.float32),
            pltpu.VMEM((RPS,), jnp.float32),
            pltpu.VMEM_SHARED((NP, D), jnp.float32),
            pltpu.VMEM_SHARED((NP,), jnp.float32),
            pltpu.SemaphoreType.DMA,
            pltpu.SemaphoreType.DMA,
            pltpu.SemaphoreType.DMA,
            pltpu.SemaphoreType.DMA,
            pltpu.SemaphoreType.DMA,
            pltpu.SemaphoreType.DMA,
        ],
    )
    def scatter_k(dh_hbm, ei_hbm, s_out, cnt0_out, cnt1_out,
                  idx_v, r0, r1, ones_v, zcnt_v, acc_sh, cnt_sh,
                  sl0, sl1, ss0, ss1, so0, so1):
        cid = lax.axis_index("c")
        sid = lax.axis_index("s")
        wid = sid * NC + cid
        base = wid * EPW

        idx_cp = pltpu.async_copy(ei_hbm.at[pl.ds(base, EPW)], idx_v, sl0)

        zeros16 = jnp.zeros((LANES,), jnp.float32)
        ones16 = jnp.ones((LANES,), jnp.float32)

        def zrow(c, carry):
            for j in range(D // LANES):
                r0[c, pl.ds(j * LANES, LANES)] = zeros16
            return carry

        lax.fori_loop(0, C, zrow, 0, unroll=2)
        for j in range(C // LANES):
            ones_v[pl.ds(j * LANES, LANES)] = ones16

        def zc(i, carry):
            zcnt_v[pl.ds(i * LANES, LANES)] = zeros16
            return carry

        lax.fori_loop(0, RPS // LANES, zc, 0, unroll=2)

        row0 = sid * RPS

        def zbody(k, carry):
            pltpu.sync_copy(r0, acc_sh.at[pl.ds(row0 + k * C, C)])
            return carry

        lax.fori_loop(0, RPS // C, zbody, 0)
        pltpu.sync_copy(zcnt_v, cnt_sh.at[pl.ds(row0, RPS)])
        idx_cp.wait()
        plsc.subcore_barrier()

        def fire_load(c, rbuf, sl):
            pltpu.async_copy(dh_hbm.at[pl.ds(base + c * C, C)], rbuf, sl)

        def wait_load(rbuf, sl):
            pltpu.make_async_copy(dh_hbm.at[pl.ds(base, C)], rbuf, sl).wait()

        def wait_scat(c, rbuf, ss, so):
            isl = idx_v.at[pl.ds(c * C, C)]
            pltpu.make_async_copy(rbuf, acc_sh.at[isl], ss).wait()
            pltpu.make_async_copy(ones_v, cnt_sh.at[isl], so).wait()

        def step(c, rbuf, sl, ss, so):
            wait_load(rbuf, sl)
            isl = idx_v.at[pl.ds(c * C, C)]
            pltpu.async_copy(rbuf, acc_sh.at[isl], ss, add=True)
            pltpu.async_copy(ones_v, cnt_sh.at[isl], so, add=True)

            @pl.when(c + 2 < NCHUNK)
            def _():
                wait_scat(c, rbuf, ss, so)
                fire_load(c + 2, rbuf, sl)

        fire_load(0, r0, sl0)
        fire_load(1, r1, sl1)

        def pair(i2, carry):
            c0 = 2 * i2
            step(c0, r0, sl0, ss0, so0)
            step(c0 + 1, r1, sl1, ss1, so1)
            return carry

        lax.fori_loop(0, NCHUNK // 2, pair, 0)

        if NCHUNK % 2 == 1:
            step(NCHUNK - 1, r0, sl0, ss0, so0)

        wait_scat(NCHUNK - 1, r0, ss0, so0)
        if NCHUNK >= 2:
            wait_scat(NCHUNK - 2, r1, ss1, so1)
        plsc.subcore_barrier()

        pltpu.sync_copy(acc_sh.at[pl.ds(row0, RPS)],
                        s_out.at[cid, pl.ds(row0, RPS)])

        @pl.when(cid == 0)
        def _():
            pltpu.sync_copy(cnt_sh.at[pl.ds(row0, RPS)],
                            cnt0_out.at[pl.ds(row0, RPS)])

        @pl.when(cid == 1)
        def _():
            pltpu.sync_copy(cnt_sh.at[pl.ds(row0, RPS)],
                            cnt1_out.at[pl.ds(row0, RPS)])

    return scatter_k


def _final_body(hv_ref, s_ref, c0_ref, c1_ref, w3_ref, b3_ref, dw1_ref, db1_ref,
                dw2_ref, db2_ref, g0_ref, be0_ref, g1_ref, be1_ref, o_ref):
    n = hv_ref.shape[0]
    s = s_ref[0, :n, :] + s_ref[1, :n, :]
    cnt = c0_ref[:n] + c1_ref[:n]
    m = s / jnp.maximum(cnt, 1.0)[:, None]
    dh = jnp.dot(m, w3_ref[...], preferred_element_type=jnp.float32) + b3_ref[...]
    t = hv_ref[...] + dh
    mu = jnp.mean(t, axis=0)
    var = jnp.mean((t - mu) ** 2, axis=0)
    h = (t - mu) * lax.rsqrt(var + BN_EPS) * g0_ref[...] + be0_ref[...]
    u = _silu(jnp.dot(h, dw1_ref[...], preferred_element_type=jnp.float32)
              + db1_ref[...])
    d2 = jnp.dot(u, dw2_ref[...], preferred_element_type=jnp.float32) + db2_ref[...]
    t2 = h + d2
    mu2 = jnp.mean(t2, axis=0)
    var2 = jnp.mean((t2 - mu2) ** 2, axis=0)
    o_ref[...] = (t2 - mu2) * lax.rsqrt(var2 + BN_EPS) * g1_ref[...] + be1_ref[...]


def _final(h_V, s2, c0, c1, W3, b3, Dw1, Db1, Dw2, Db2, g0, be0, g1, be1):
    n, d = h_V.shape
    return pl.pallas_call(
        _final_body,
        out_shape=jax.ShapeDtypeStruct((n, d), jnp.float32),
    )(h_V, s2, c0, c1, W3, b3, Dw1, Db1, Dw2, Db2, g0, be0, g1, be1)


def kernel(h_V, edge_idx, batch_id, W1, b1, W2, b2, W3, b3,
           Dw1, Db1, Dw2, Db2, g0, be0, g1, be1):
    n, d = h_V.shape
    E = edge_idx.shape[1]

    NP = -(-n // (NS * LANES)) * (NS * LANES)
    C = 80

    ei = edge_idx.reshape(2 * E)
    A, B = _nodeproj(h_V, W1[:d], W1[d:], b1)
    G = _make_gather(E, d, C)(A, B, ei)
    dh2 = _edgemlp(G, W2, b2, BE=4000)
    s2, c0, c1 = _make_scatter(E, d, C, NP)(dh2, ei)
    return _final(h_V, s2, c0, c1, W3, b3,
                  Dw1, Db1, Dw2, Db2, g0, be0, g1, be1)
